# baseline (speedup 1.0000x reference)
import functools

import jax
import jax.numpy as jnp
from jax import lax
from jax.experimental import pallas as pl
from jax.experimental.pallas import tpu as pltpu
from jax.experimental.pallas import tpu_sc as plsc

D = 32
L2_REG = 1e-4
B = 16384

NC = 2
NS = 16
L = 16
NW = NC * NS
BPW = B // NW
CHUNK = 128
NCHUNK = BPW // CHUNK


def _sc_body(uidx, pidx, nidx, uemb, iemb, pvec_out, reg_out,
             idx_u, idx_p, idx_n, rows_u, rows_p, rows_n, pout, regbuf, sem):
    wid = lax.axis_index("s") * NC + lax.axis_index("c")
    base = wid * BPW

    pltpu.sync_copy(uidx.at[pl.ds(base, BPW)], idx_u)
    pltpu.sync_copy(pidx.at[pl.ds(base, BPW)], idx_p)
    pltpu.sync_copy(nidx.at[pl.ds(base, BPW)], idx_n)

    descs = []
    for j in range(NCHUNK):
        sl = pl.ds(j * CHUNK, CHUNK)
        descs.append(pltpu.async_copy(uemb.at[idx_u.at[sl]], rows_u.at[sl], sem))
        descs.append(pltpu.async_copy(iemb.at[idx_p.at[sl]], rows_p.at[sl], sem))
        descs.append(pltpu.async_copy(iemb.at[idx_n.at[sl]], rows_n.at[sl], sem))
    for d in descs:
        d.wait()

    def row_step(r, regacc):
        u0 = rows_u[r, pl.ds(0, L)]
        u1 = rows_u[r, pl.ds(L, L)]
        p0 = rows_p[r, pl.ds(0, L)]
        p1 = rows_p[r, pl.ds(L, L)]
        n0 = rows_n[r, pl.ds(0, L)]
        n1 = rows_n[r, pl.ds(L, L)]
        pout[r, pl.ds(0, L)] = u0 * (p0 - n0) + u1 * (p1 - n1)
        return regacc + (u0 * u0 + u1 * u1 + p0 * p0 + p1 * p1
                         + n0 * n0 + n1 * n1)

    regv = lax.fori_loop(0, BPW, row_step, jnp.zeros((L,), jnp.float32))

    pltpu.sync_copy(pout, pvec_out.at[pl.ds(base, BPW)])
    regbuf[pl.ds(0, L)] = regv
    pltpu.sync_copy(regbuf, reg_out.at[wid])


@jax.jit
def _sc_call(uidx, pidx, nidx, uemb, iemb):
    mesh = plsc.VectorSubcoreMesh(core_axis_name="c", subcore_axis_name="s",
                                  num_cores=NC, num_subcores=NS)
    f = functools.partial(
        pl.kernel,
        out_type=(jax.ShapeDtypeStruct((B, L), jnp.float32),
                  jax.ShapeDtypeStruct((NW, L), jnp.float32)),
        mesh=mesh,
        compiler_params=pltpu.CompilerParams(use_tc_tiling_on_sc=False),
        scratch_types=[
            pltpu.VMEM((BPW,), jnp.int32),
            pltpu.VMEM((BPW,), jnp.int32),
            pltpu.VMEM((BPW,), jnp.int32),
            pltpu.VMEM((BPW, D), jnp.float32),
            pltpu.VMEM((BPW, D), jnp.float32),
            pltpu.VMEM((BPW, D), jnp.float32),
            pltpu.VMEM((BPW, L), jnp.float32),
            pltpu.VMEM((L,), jnp.float32),
            pltpu.SemaphoreType.DMA,
        ],
    )(_sc_body)
    return f(uidx, pidx, nidx, uemb, iemb)


def _tc_body(pvec_ref, reg_ref, out_ref):
    x = pvec_ref[...]
    c = lax.broadcasted_iota(jnp.int32, (128, 8), 0)
    j = lax.broadcasted_iota(jnp.int32, (128, 8), 1)
    sel = jnp.where(c // L == j, 1.0, 0.0).astype(jnp.float32)
    diff = jax.lax.dot_general(x, sel, (((1,), (0,)), ((), ())),
                               preferred_element_type=jnp.float32)
    loss = -jnp.mean(jnp.log(jax.nn.sigmoid(diff) + 1e-8))
    reg = jnp.sum(reg_ref[...])
    out_ref[0, 0] = loss + L2_REG * (reg / B)


@jax.jit
def _tc_call(pvec, regpart):
    out = pl.pallas_call(
        _tc_body,
        out_shape=jax.ShapeDtypeStruct((1, 1), jnp.float32),
        out_specs=pl.BlockSpec(memory_space=pltpu.SMEM),
    )(pvec.reshape(2048, 128), regpart.reshape(4, 128))
    return out.reshape(())


def kernel(user_indices, pos_item_indices, neg_item_indices,
           user_embedding, item_embedding):
    pvec, regpart = _sc_call(user_indices, pos_item_indices, neg_item_indices,
                             user_embedding, item_embedding)
    return _tc_call(pvec, regpart)
